# baseline (speedup 1.0000x reference)
import functools

import jax
import jax.numpy as jnp
from jax import lax
from jax.experimental import pallas as pl
from jax.experimental.pallas import tpu as pltpu
from jax.experimental.pallas import tpu_sc as plsc

N_ON = 647
N_OFF = 651
N_R = N_ON + N_OFF
N_GATHER = 1947
N_L = N_R + N_GATHER
D = 12544
N_V1 = 784


_C_R = 896


def _r_body(x_ref, on_ref, off_ref, r_ref):
    xv = x_ref[0:1, :]
    r_ref[0:N_ON, :] = on_ref[...] * xv
    r_ref[N_ON:N_R, :] = off_ref[...] * (1.0 - xv)


def _compute_r(x2, on2, off2):
    return pl.pallas_call(
        _r_body,
        grid=(D // _C_R,),
        in_specs=[
            pl.BlockSpec((1, _C_R), lambda j: (0, j)),
            pl.BlockSpec((N_ON, _C_R), lambda j: (0, j)),
            pl.BlockSpec((N_OFF, _C_R), lambda j: (0, j)),
        ],
        out_specs=pl.BlockSpec((N_R, _C_R), lambda j: (0, j)),
        out_shape=jax.ShapeDtypeStruct((N_R, D), jnp.float32),
    )(x2, on2, off2)


def _v_body(x_ref, on_ref, off_ref, cf_ref, v_ref):
    xv = x_ref[0:1, :]
    ron = (on_ref[...] * xv).astype(jnp.bfloat16)
    roff = (off_ref[...] * (1.0 - xv)).astype(jnp.bfloat16)
    v_ref[...] = (
        jnp.dot(cf_ref[:, 0:N_ON], ron, preferred_element_type=jnp.float32)
        + jnp.dot(cf_ref[:, N_ON:N_R], roff, preferred_element_type=jnp.float32))


def _compute_v(x2, on2, off2, conn_fold):
    return pl.pallas_call(
        _v_body,
        grid=(D // _C_R,),
        in_specs=[
            pl.BlockSpec((1, _C_R), lambda j: (0, j)),
            pl.BlockSpec((N_ON, _C_R), lambda j: (0, j)),
            pl.BlockSpec((N_OFF, _C_R), lambda j: (0, j)),
            pl.BlockSpec((N_V1, N_R), lambda j: (0, 0)),
        ],
        out_specs=pl.BlockSpec((N_V1, _C_R), lambda j: (0, j)),
        out_shape=jax.ShapeDtypeStruct((N_V1, D), jnp.float32),
    )(x2, on2, off2, conn_fold)


_NW = 32
_GCHUNK = 104
_GPAD = _NW * _GCHUNK
_W = 8
_NWIN = _GCHUNK // _W


def _l_body(gidx_hbm, r_hbm, l_hbm, idx_v, src_slot, dst_slot, buf,
            sem_g, sem_s):
    wid = lax.axis_index("s") * 2 + lax.axis_index("c")
    base = wid * _GCHUNK
    pltpu.sync_copy(gidx_hbm.at[pl.ds(base, _GCHUNK)], idx_v)
    lane = lax.iota(jnp.int32, 16)
    lane_mask = lane < _W
    lane_c = jnp.minimum(lane, _W - 1)

    for t in range(_NWIN):
        start = base + t * _W

        @pl.when(start < N_L)
        def _():
            offs = jnp.minimum(t * _W + lane, _GCHUNK - 1)
            src_slot[...] = plsc.load_gather(idx_v, [offs])
            dst16 = jnp.minimum(start + lane, N_L - 1)
            plsc.store_scatter(dst_slot, [lane_c], dst16, mask=lane_mask)
            pltpu.async_copy(r_hbm.at[src_slot.at[pl.ds(0, _W)]], buf,
                             sem_g).wait()
            pltpu.async_copy(buf, l_hbm.at[dst_slot], sem_s).wait()


def _compute_l(gather_idx, r):
    f = functools.partial(
        pl.kernel,
        out_type=jax.ShapeDtypeStruct((N_L, D), jnp.float32),
        mesh=plsc.VectorSubcoreMesh(core_axis_name="c", subcore_axis_name="s",
                                    num_cores=2, num_subcores=16),
        compiler_params=pltpu.CompilerParams(needs_layout_passes=False),
        scratch_types=[
            pltpu.VMEM((_GCHUNK,), jnp.int32),
            pltpu.VMEM((16,), jnp.int32),
            pltpu.VMEM((_W,), jnp.int32),
            pltpu.VMEM((_W, D), jnp.float32),
            pltpu.SemaphoreType.DMA,
            pltpu.SemaphoreType.DMA,
        ],
    )(_l_body)
    return f(gather_idx, r)


def _fold_body(conn1_ref, conn2_ref, idx_ref, out_ref):
    cols = lax.broadcasted_iota(jnp.int32, (N_GATHER, N_R), 1)
    onehot = (cols == idx_ref[...]).astype(jnp.float32)
    out_ref[...] = (conn1_ref[...] + jnp.dot(
        conn2_ref[...], onehot,
        preferred_element_type=jnp.float32)).astype(jnp.bfloat16)


def _compute_fold(conn1, conn2, idx2d):
    return pl.pallas_call(
        _fold_body,
        in_specs=[
            pl.BlockSpec((N_V1, N_R), lambda: (0, 0)),
            pl.BlockSpec((N_V1, N_GATHER), lambda: (0, 0)),
            pl.BlockSpec((N_GATHER, 1), lambda: (0, 0)),
        ],
        out_specs=pl.BlockSpec((N_V1, N_R), lambda: (0, 0)),
        out_shape=jax.ShapeDtypeStruct((N_V1, N_R), jnp.bfloat16),
    )(conn1, conn2, idx2d)


def kernel(x, acts_on, acts_off, lgn_idx, conn):
    x2 = x.reshape(1, D)
    on2 = acts_on.reshape(N_ON, D)
    off2 = acts_off.reshape(N_OFF, D)
    gidx = jnp.concatenate([jnp.arange(N_R, dtype=jnp.int32),
                            lgn_idx.astype(jnp.int32)])
    gather_idx = jnp.concatenate([
        gidx,
        jnp.broadcast_to(gidx[N_L - 1:], (_GPAD - N_L,)),
    ])
    conn_fold = _compute_fold(conn[:, :N_R], conn[:, N_R:],
                              lgn_idx.astype(jnp.int32).reshape(N_GATHER, 1))
    r = _compute_r(x2, on2, off2)
    l = _compute_l(gather_idx, r)
    v = _compute_v(x2, on2, off2, conn_fold)
    return (r, l, v)


# speedup vs baseline: 1.0046x; 1.0046x over previous
import functools

import jax
import jax.numpy as jnp
from jax import lax
from jax.experimental import pallas as pl
from jax.experimental.pallas import tpu as pltpu
from jax.experimental.pallas import tpu_sc as plsc

N_ON = 647
N_OFF = 651
N_R = N_ON + N_OFF
N_GATHER = 1947
N_L = N_R + N_GATHER
D = 12544
N_V1 = 784


_C_R = 896


def _rv_body(x_ref, on_ref, off_ref, cf_ref, r_ref, v_ref):
    xv = x_ref[0:1, :]
    r_ref[0:N_ON, :] = on_ref[...] * xv
    r_ref[N_ON:N_R, :] = off_ref[...] * (1.0 - xv)
    v_ref[...] = jnp.dot(cf_ref[...], r_ref[...].astype(jnp.bfloat16),
                         preferred_element_type=jnp.float32)


def _compute_rv(x2, on2, off2, conn_fold):
    return pl.pallas_call(
        _rv_body,
        grid=(D // _C_R,),
        in_specs=[
            pl.BlockSpec((1, _C_R), lambda j: (0, j)),
            pl.BlockSpec((N_ON, _C_R), lambda j: (0, j)),
            pl.BlockSpec((N_OFF, _C_R), lambda j: (0, j)),
            pl.BlockSpec((N_V1, N_R), lambda j: (0, 0)),
        ],
        out_specs=[
            pl.BlockSpec((N_R, _C_R), lambda j: (0, j)),
            pl.BlockSpec((N_V1, _C_R), lambda j: (0, j)),
        ],
        out_shape=[
            jax.ShapeDtypeStruct((N_R, D), jnp.float32),
            jax.ShapeDtypeStruct((N_V1, D), jnp.float32),
        ],
    )(x2, on2, off2, conn_fold)


_NW = 32
_GCHUNK = 104
_GPAD = _NW * _GCHUNK
_W = 8
_NWIN = _GCHUNK // _W


def _l_body(gidx_hbm, r_hbm, l_hbm, idx_v, src_slot, dst_slot, buf,
            sem_g, sem_s):
    wid = lax.axis_index("s") * 2 + lax.axis_index("c")
    base = wid * _GCHUNK
    pltpu.sync_copy(gidx_hbm.at[pl.ds(base, _GCHUNK)], idx_v)
    lane = lax.iota(jnp.int32, 16)
    lane_mask = lane < _W
    lane_c = jnp.minimum(lane, _W - 1)

    for t in range(_NWIN):
        start = base + t * _W

        @pl.when(start < N_L)
        def _():
            offs = jnp.minimum(t * _W + lane, _GCHUNK - 1)
            src_slot[...] = plsc.load_gather(idx_v, [offs])
            dst16 = jnp.minimum(start + lane, N_L - 1)
            plsc.store_scatter(dst_slot, [lane_c], dst16, mask=lane_mask)
            pltpu.async_copy(r_hbm.at[src_slot.at[pl.ds(0, _W)]], buf,
                             sem_g).wait()
            pltpu.async_copy(buf, l_hbm.at[dst_slot], sem_s).wait()


def _compute_l(gather_idx, r):
    f = functools.partial(
        pl.kernel,
        out_type=jax.ShapeDtypeStruct((N_L, D), jnp.float32),
        mesh=plsc.VectorSubcoreMesh(core_axis_name="c", subcore_axis_name="s",
                                    num_cores=2, num_subcores=16),
        compiler_params=pltpu.CompilerParams(needs_layout_passes=False),
        scratch_types=[
            pltpu.VMEM((_GCHUNK,), jnp.int32),
            pltpu.VMEM((16,), jnp.int32),
            pltpu.VMEM((_W,), jnp.int32),
            pltpu.VMEM((_W, D), jnp.float32),
            pltpu.SemaphoreType.DMA,
            pltpu.SemaphoreType.DMA,
        ],
    )(_l_body)
    return f(gather_idx, r)


def _fold_body(conn1_ref, conn2_ref, idx_ref, out_ref):
    cols = lax.broadcasted_iota(jnp.int32, (N_GATHER, N_R), 1)
    onehot = (cols == idx_ref[...]).astype(jnp.float32)
    out_ref[...] = (conn1_ref[...] + jnp.dot(
        conn2_ref[...], onehot,
        preferred_element_type=jnp.float32)).astype(jnp.bfloat16)


def _compute_fold(conn1, conn2, idx2d):
    return pl.pallas_call(
        _fold_body,
        in_specs=[
            pl.BlockSpec((N_V1, N_R), lambda: (0, 0)),
            pl.BlockSpec((N_V1, N_GATHER), lambda: (0, 0)),
            pl.BlockSpec((N_GATHER, 1), lambda: (0, 0)),
        ],
        out_specs=pl.BlockSpec((N_V1, N_R), lambda: (0, 0)),
        out_shape=jax.ShapeDtypeStruct((N_V1, N_R), jnp.bfloat16),
    )(conn1, conn2, idx2d)


def kernel(x, acts_on, acts_off, lgn_idx, conn):
    x2 = x.reshape(1, D)
    on2 = acts_on.reshape(N_ON, D)
    off2 = acts_off.reshape(N_OFF, D)
    gidx = jnp.concatenate([jnp.arange(N_R, dtype=jnp.int32),
                            lgn_idx.astype(jnp.int32)])
    gather_idx = jnp.concatenate([
        gidx,
        jnp.broadcast_to(gidx[N_L - 1:], (_GPAD - N_L,)),
    ])
    conn_fold = _compute_fold(conn[:, :N_R], conn[:, N_R:],
                              lgn_idx.astype(jnp.int32).reshape(N_GATHER, 1))
    r, v = _compute_rv(x2, on2, off2, conn_fold)
    l = _compute_l(gather_idx, r)
    return (r, l, v)


# speedup vs baseline: 1.0461x; 1.0413x over previous
import functools

import jax
import jax.numpy as jnp
from jax import lax
from jax.experimental import pallas as pl
from jax.experimental.pallas import tpu as pltpu
from jax.experimental.pallas import tpu_sc as plsc

N_ON = 647
N_OFF = 651
N_R = N_ON + N_OFF
N_GATHER = 1947
N_L = N_R + N_GATHER
D = 12544
N_V1 = 784


_C_R = 896


def _rv_body(x_ref, on_ref, off_ref, cf_ref, r_ref, v_ref):
    xv = x_ref[0:1, :]
    r_ref[0:N_ON, :] = on_ref[...] * xv
    r_ref[N_ON:N_R, :] = off_ref[...] * (1.0 - xv)
    v_ref[...] = jnp.dot(cf_ref[...], r_ref[...].astype(jnp.bfloat16),
                         preferred_element_type=jnp.float32)


def _compute_rv(x2, on2, off2, conn_fold):
    return pl.pallas_call(
        _rv_body,
        grid=(D // _C_R,),
        in_specs=[
            pl.BlockSpec((1, _C_R), lambda j: (0, j)),
            pl.BlockSpec((N_ON, _C_R), lambda j: (0, j)),
            pl.BlockSpec((N_OFF, _C_R), lambda j: (0, j)),
            pl.BlockSpec((N_V1, N_R), lambda j: (0, 0)),
        ],
        out_specs=[
            pl.BlockSpec((N_R, _C_R), lambda j: (0, j)),
            pl.BlockSpec((N_V1, _C_R), lambda j: (0, j)),
        ],
        out_shape=[
            jax.ShapeDtypeStruct((N_R, D), jnp.float32),
            jax.ShapeDtypeStruct((N_V1, D), jnp.float32),
        ],
    )(x2, on2, off2, conn_fold)


_NW = 32
_GCHUNK = 104
_GPAD = _NW * _GCHUNK
_W = 8
_NWIN = _GCHUNK // _W


def _l_body(gidx_hbm, r_hbm, l_hbm, idx_v, src_slot, dst_slot, buf,
            sem_g, sem_s):
    wid = lax.axis_index("s") * 2 + lax.axis_index("c")
    base = wid * _GCHUNK
    pltpu.sync_copy(gidx_hbm.at[pl.ds(base, _GCHUNK)], idx_v)
    lane = lax.iota(jnp.int32, 16)
    lane_mask = lane < _W
    lane_c = jnp.minimum(lane, _W - 1)

    for t in range(_NWIN):
        start = base + t * _W

        @pl.when(start < N_L)
        def _():
            offs = jnp.minimum(t * _W + lane, _GCHUNK - 1)
            src_slot[...] = plsc.load_gather(idx_v, [offs])
            dst16 = jnp.minimum(start + lane, N_L - 1)
            plsc.store_scatter(dst_slot, [lane_c], dst16, mask=lane_mask)
            pltpu.async_copy(r_hbm.at[src_slot.at[pl.ds(0, _W)]], buf,
                             sem_g).wait()
            pltpu.async_copy(buf, l_hbm.at[dst_slot], sem_s).wait()


def _compute_l(gather_idx, r):
    f = functools.partial(
        pl.kernel,
        out_type=jax.ShapeDtypeStruct((N_L, D), jnp.float32),
        mesh=plsc.VectorSubcoreMesh(core_axis_name="c", subcore_axis_name="s",
                                    num_cores=2, num_subcores=16),
        compiler_params=pltpu.CompilerParams(needs_layout_passes=False),
        scratch_types=[
            pltpu.VMEM((_GCHUNK,), jnp.int32),
            pltpu.VMEM((16,), jnp.int32),
            pltpu.VMEM((_W,), jnp.int32),
            pltpu.VMEM((_W, D), jnp.float32),
            pltpu.SemaphoreType.DMA,
            pltpu.SemaphoreType.DMA,
        ],
    )(_l_body)
    return f(gather_idx, r)


def _fold_body(conn_ref, idx_ref, out_ref):
    cols = lax.broadcasted_iota(jnp.int32, (N_GATHER, N_R), 1)
    onehot = (cols == idx_ref[...]).astype(jnp.float32)
    out_ref[...] = (conn_ref[:, 0:N_R] + jnp.dot(
        conn_ref[:, N_R:N_L], onehot,
        preferred_element_type=jnp.float32)).astype(jnp.bfloat16)


def _compute_fold(conn, idx2d):
    return pl.pallas_call(
        _fold_body,
        in_specs=[
            pl.BlockSpec((N_V1, N_L), lambda: (0, 0)),
            pl.BlockSpec((N_GATHER, 1), lambda: (0, 0)),
        ],
        out_specs=pl.BlockSpec((N_V1, N_R), lambda: (0, 0)),
        out_shape=jax.ShapeDtypeStruct((N_V1, N_R), jnp.bfloat16),
    )(conn, idx2d)


def kernel(x, acts_on, acts_off, lgn_idx, conn):
    x2 = x.reshape(1, D)
    on2 = acts_on.reshape(N_ON, D)
    off2 = acts_off.reshape(N_OFF, D)
    gidx = jnp.concatenate([jnp.arange(N_R, dtype=jnp.int32),
                            lgn_idx.astype(jnp.int32)])
    gather_idx = jnp.concatenate([
        gidx,
        jnp.broadcast_to(gidx[N_L - 1:], (_GPAD - N_L,)),
    ])
    conn_fold = _compute_fold(conn,
                              lgn_idx.astype(jnp.int32).reshape(N_GATHER, 1))
    r, v = _compute_rv(x2, on2, off2, conn_fold)
    l = _compute_l(gather_idx, r)
    return (r, l, v)


# speedup vs baseline: 1.0497x; 1.0035x over previous
import functools

import jax
import jax.numpy as jnp
from jax import lax
from jax.experimental import pallas as pl
from jax.experimental.pallas import tpu as pltpu
from jax.experimental.pallas import tpu_sc as plsc

N_ON = 647
N_OFF = 651
N_R = N_ON + N_OFF
N_GATHER = 1947
N_L = N_R + N_GATHER
D = 12544
N_V1 = 784


_C_R = 1792


def _rv_body(x_ref, on_ref, off_ref, cf_ref, r_ref, v_ref):
    xv = x_ref[0:1, :]
    r_ref[0:N_ON, :] = on_ref[...] * xv
    r_ref[N_ON:N_R, :] = off_ref[...] * (1.0 - xv)
    v_ref[...] = jnp.dot(cf_ref[...], r_ref[...].astype(jnp.bfloat16),
                         preferred_element_type=jnp.float32)


def _compute_rv(x2, on2, off2, conn_fold):
    return pl.pallas_call(
        _rv_body,
        grid=(D // _C_R,),
        in_specs=[
            pl.BlockSpec((1, _C_R), lambda j: (0, j)),
            pl.BlockSpec((N_ON, _C_R), lambda j: (0, j)),
            pl.BlockSpec((N_OFF, _C_R), lambda j: (0, j)),
            pl.BlockSpec((N_V1, N_R), lambda j: (0, 0)),
        ],
        out_specs=[
            pl.BlockSpec((N_R, _C_R), lambda j: (0, j)),
            pl.BlockSpec((N_V1, _C_R), lambda j: (0, j)),
        ],
        out_shape=[
            jax.ShapeDtypeStruct((N_R, D), jnp.float32),
            jax.ShapeDtypeStruct((N_V1, D), jnp.float32),
        ],
    )(x2, on2, off2, conn_fold)


_NW = 32
_GCHUNK = 104
_GPAD = _NW * _GCHUNK
_W = 8
_NWIN = _GCHUNK // _W


def _l_body(gidx_hbm, r_hbm, l_hbm, idx_v, src_slot, dst_slot, buf,
            sem_g, sem_s):
    wid = lax.axis_index("s") * 2 + lax.axis_index("c")
    base = wid * _GCHUNK
    pltpu.sync_copy(gidx_hbm.at[pl.ds(base, _GCHUNK)], idx_v)
    lane = lax.iota(jnp.int32, 16)
    lane_mask = lane < _W
    lane_c = jnp.minimum(lane, _W - 1)

    for t in range(_NWIN):
        start = base + t * _W

        @pl.when(start < N_L)
        def _():
            offs = jnp.minimum(t * _W + lane, _GCHUNK - 1)
            src_slot[...] = plsc.load_gather(idx_v, [offs])
            dst16 = jnp.minimum(start + lane, N_L - 1)
            plsc.store_scatter(dst_slot, [lane_c], dst16, mask=lane_mask)
            pltpu.async_copy(r_hbm.at[src_slot.at[pl.ds(0, _W)]], buf,
                             sem_g).wait()
            pltpu.async_copy(buf, l_hbm.at[dst_slot], sem_s).wait()


def _compute_l(gather_idx, r):
    f = functools.partial(
        pl.kernel,
        out_type=jax.ShapeDtypeStruct((N_L, D), jnp.float32),
        mesh=plsc.VectorSubcoreMesh(core_axis_name="c", subcore_axis_name="s",
                                    num_cores=2, num_subcores=16),
        compiler_params=pltpu.CompilerParams(needs_layout_passes=False),
        scratch_types=[
            pltpu.VMEM((_GCHUNK,), jnp.int32),
            pltpu.VMEM((16,), jnp.int32),
            pltpu.VMEM((_W,), jnp.int32),
            pltpu.VMEM((_W, D), jnp.float32),
            pltpu.SemaphoreType.DMA,
            pltpu.SemaphoreType.DMA,
        ],
    )(_l_body)
    return f(gather_idx, r)


def _fold_body(conn_ref, idx_ref, out_ref):
    cols = lax.broadcasted_iota(jnp.int32, (N_GATHER, N_R), 1)
    onehot = (cols == idx_ref[...]).astype(jnp.float32)
    out_ref[...] = (conn_ref[:, 0:N_R] + jnp.dot(
        conn_ref[:, N_R:N_L], onehot,
        preferred_element_type=jnp.float32)).astype(jnp.bfloat16)


def _compute_fold(conn, idx2d):
    return pl.pallas_call(
        _fold_body,
        in_specs=[
            pl.BlockSpec((N_V1, N_L), lambda: (0, 0)),
            pl.BlockSpec((N_GATHER, 1), lambda: (0, 0)),
        ],
        out_specs=pl.BlockSpec((N_V1, N_R), lambda: (0, 0)),
        out_shape=jax.ShapeDtypeStruct((N_V1, N_R), jnp.bfloat16),
    )(conn, idx2d)


def kernel(x, acts_on, acts_off, lgn_idx, conn):
    x2 = x.reshape(1, D)
    on2 = acts_on.reshape(N_ON, D)
    off2 = acts_off.reshape(N_OFF, D)
    gidx = jnp.concatenate([jnp.arange(N_R, dtype=jnp.int32),
                            lgn_idx.astype(jnp.int32)])
    gather_idx = jnp.concatenate([
        gidx,
        jnp.broadcast_to(gidx[N_L - 1:], (_GPAD - N_L,)),
    ])
    conn_fold = _compute_fold(conn,
                              lgn_idx.astype(jnp.int32).reshape(N_GATHER, 1))
    r, v = _compute_rv(x2, on2, off2, conn_fold)
    l = _compute_l(gather_idx, r)
    return (r, l, v)
